# Initial kernel scaffold; baseline (speedup 1.0000x reference)
import jax
import jax.numpy as jnp
from jax.experimental import pallas as pl


def kernel(positions, query, key):
    raise NotImplementedError("write your pallas kernel here")


# speedup vs baseline: 4.9018x; 4.9018x over previous
import functools
import math

import jax
import jax.numpy as jnp
from jax import lax
from jax.experimental import pallas as pl
from jax.experimental.pallas import tpu as pltpu
from jax.experimental.pallas import tpu_sc as plsc

_HEAD = 128
_HALF = 64
_MAX_POS = 8192
_ORIG_MAX_POS = 4096
_BASE = 10000.0
_SCALE = math.sqrt(1.0 + math.log(_MAX_POS / _ORIG_MAX_POS) / math.log(_ORIG_MAX_POS))


def _cs_table():
    inv_freq = 1.0 / (_BASE ** (jnp.arange(0, _HEAD, 2, dtype=jnp.float32) / _HEAD))
    t = jnp.arange(_MAX_POS, dtype=jnp.float32)
    freqs = jnp.outer(t, inv_freq)
    return jnp.concatenate([jnp.cos(freqs), jnp.sin(freqs)], axis=-1) * _SCALE


def _sc_gather(table, positions):
    n = positions.shape[0]
    info = plsc.get_sparse_core_info()
    ncores, nsub = info.num_cores, info.num_subcores
    nw = ncores * nsub
    chunks = n // (nw * 128)
    pos2 = positions.reshape(n // 128, 128).astype(jnp.int32)

    mesh = plsc.VectorSubcoreMesh(core_axis_name="c", subcore_axis_name="s")

    @functools.partial(
        pl.kernel,
        mesh=mesh,
        out_type=jax.ShapeDtypeStruct((n, _HEAD), jnp.float32),
        scratch_types=[
            pltpu.VMEM((chunks, 128), jnp.int32),
            pltpu.VMEM((chunks * 128, _HEAD), jnp.float32),
            pltpu.SemaphoreType.DMA,
        ],
    )
    def gather_k(table_hbm, pos_hbm, out_hbm, idx_v, rows_v, sem):
        wid = lax.axis_index("s") * ncores + lax.axis_index("c")
        row0 = wid * chunks
        pltpu.sync_copy(pos_hbm.at[pl.ds(row0, chunks)], idx_v)
        copies = [
            pltpu.async_copy(
                table_hbm.at[idx_v.at[j]], rows_v.at[pl.ds(j * 128, 128)], sem
            )
            for j in range(chunks)
        ]
        for c in copies:
            c.wait()
        pltpu.sync_copy(rows_v, out_hbm.at[pl.ds(row0 * 128, chunks * 128)])

    return gather_k(table, pos2)


def _tc_apply(cs, query, key, block_t=256):
    n, hidden = query.shape
    heads = hidden // _HEAD

    def body(cs_ref, q_ref, k_ref, oq_ref, ok_ref):
        c = cs_ref[:, :_HALF]
        s = cs_ref[:, _HALF:]
        for ref, out in ((q_ref, oq_ref), (k_ref, ok_ref)):
            for h in range(heads):
                x1 = ref[:, h * _HEAD : h * _HEAD + _HALF]
                x2 = ref[:, h * _HEAD + _HALF : (h + 1) * _HEAD]
                out[:, h * _HEAD : (h + 1) * _HEAD] = jnp.concatenate(
                    [x1 * c - x2 * s, x2 * c + x1 * s], axis=1
                )

    bs = pl.BlockSpec
    return pl.pallas_call(
        body,
        grid=(n // block_t,),
        in_specs=[
            bs((block_t, _HEAD), lambda i: (i, 0)),
            bs((block_t, hidden), lambda i: (i, 0)),
            bs((block_t, hidden), lambda i: (i, 0)),
        ],
        out_specs=[
            bs((block_t, hidden), lambda i: (i, 0)),
            bs((block_t, hidden), lambda i: (i, 0)),
        ],
        out_shape=[jax.ShapeDtypeStruct((n, hidden), jnp.float32)] * 2,
        compiler_params=pltpu.CompilerParams(dimension_semantics=("arbitrary",)),
    )(cs, query, key)


def kernel(positions, query, key):
    cs = _sc_gather(_cs_table(), positions)
    q, k = _tc_apply(cs, query, key)
    return (q, k)


# speedup vs baseline: 5.2213x; 1.0652x over previous
import functools
import math

import jax
import jax.numpy as jnp
from jax import lax
from jax.experimental import pallas as pl
from jax.experimental.pallas import tpu as pltpu
from jax.experimental.pallas import tpu_sc as plsc

_HEAD = 128
_HALF = 64
_MAX_POS = 8192
_ORIG_MAX_POS = 4096
_BASE = 10000.0
_SCALE = math.sqrt(1.0 + math.log(_MAX_POS / _ORIG_MAX_POS) / math.log(_ORIG_MAX_POS))


def _cs_table():
    inv_freq = 1.0 / (_BASE ** (jnp.arange(0, _HEAD, 2, dtype=jnp.float32) / _HEAD))
    t = jnp.arange(_MAX_POS, dtype=jnp.float32)
    freqs = jnp.outer(t, inv_freq)
    return jnp.concatenate([jnp.cos(freqs), jnp.sin(freqs)], axis=-1) * _SCALE


def _sc_gather(table, positions):
    n = positions.shape[0]
    info = plsc.get_sparse_core_info()
    ncores, nsub = info.num_cores, info.num_subcores
    nw = ncores * nsub
    chunks = n // (nw * 128)
    pos2 = positions.reshape(n // 128, 128).astype(jnp.int32)

    mesh = plsc.VectorSubcoreMesh(core_axis_name="c", subcore_axis_name="s")

    @functools.partial(
        pl.kernel,
        mesh=mesh,
        out_type=jax.ShapeDtypeStruct((n, _HEAD), jnp.float32),
        scratch_types=[
            pltpu.VMEM((chunks, 128), jnp.int32),
            pltpu.VMEM((chunks * 128, _HEAD), jnp.float32),
            pltpu.SemaphoreType.DMA,
        ],
    )
    def gather_k(table_hbm, pos_hbm, out_hbm, idx_v, rows_v, sem):
        wid = lax.axis_index("s") * ncores + lax.axis_index("c")
        row0 = wid * chunks
        pltpu.sync_copy(pos_hbm.at[pl.ds(row0, chunks)], idx_v)
        copies = [
            pltpu.async_copy(
                table_hbm.at[idx_v.at[j]], rows_v.at[pl.ds(j * 128, 128)], sem
            )
            for j in range(chunks)
        ]
        for c in copies:
            c.wait()
        pltpu.sync_copy(rows_v, out_hbm.at[pl.ds(row0 * 128, chunks * 128)])

    return gather_k(table, pos2)


def _tc_apply(cs, query, key, block_t=512):
    n, hidden = query.shape
    heads = hidden // _HEAD

    def body(cs_ref, q_ref, k_ref, oq_ref, ok_ref):
        c = cs_ref[:, :_HALF]
        s = cs_ref[:, _HALF:]
        for ref, out in ((q_ref, oq_ref), (k_ref, ok_ref)):
            for h in range(heads):
                x1 = ref[:, h * _HEAD : h * _HEAD + _HALF]
                x2 = ref[:, h * _HEAD + _HALF : (h + 1) * _HEAD]
                out[:, h * _HEAD : (h + 1) * _HEAD] = jnp.concatenate(
                    [x1 * c - x2 * s, x2 * c + x1 * s], axis=1
                )

    bs = pl.BlockSpec
    return pl.pallas_call(
        body,
        grid=(n // block_t,),
        in_specs=[
            bs((block_t, _HEAD), lambda i: (i, 0)),
            bs((block_t, hidden), lambda i: (i, 0)),
            bs((block_t, hidden), lambda i: (i, 0)),
        ],
        out_specs=[
            bs((block_t, hidden), lambda i: (i, 0)),
            bs((block_t, hidden), lambda i: (i, 0)),
        ],
        out_shape=[jax.ShapeDtypeStruct((n, hidden), jnp.float32)] * 2,
        compiler_params=pltpu.CompilerParams(dimension_semantics=("parallel",)),
    )(cs, query, key)


def kernel(positions, query, key):
    cs = _sc_gather(_cs_table(), positions)
    q, k = _tc_apply(cs, query, key)
    return (q, k)


# speedup vs baseline: 5.5126x; 1.0558x over previous
import functools
import math

import jax
import jax.numpy as jnp
from jax import lax
from jax.experimental import pallas as pl
from jax.experimental.pallas import tpu as pltpu
from jax.experimental.pallas import tpu_sc as plsc

_HEAD = 128
_HALF = 64
_MAX_POS = 8192
_ORIG_MAX_POS = 4096
_BASE = 10000.0
_SCALE = math.sqrt(1.0 + math.log(_MAX_POS / _ORIG_MAX_POS) / math.log(_ORIG_MAX_POS))


def _cs_table():
    inv_freq = 1.0 / (_BASE ** (jnp.arange(0, _HEAD, 2, dtype=jnp.float32) / _HEAD))
    t = jnp.arange(_MAX_POS, dtype=jnp.float32)
    freqs = jnp.outer(t, inv_freq)
    return jnp.concatenate([jnp.cos(freqs), jnp.sin(freqs)], axis=-1) * _SCALE


def _sc_gather(table, positions):
    n = positions.shape[0]
    info = plsc.get_sparse_core_info()
    ncores, nsub = info.num_cores, info.num_subcores
    nw = ncores * nsub
    chunks = n // (nw * 128)
    pos2 = positions.reshape(n // 128, 128).astype(jnp.int32)

    mesh = plsc.VectorSubcoreMesh(core_axis_name="c", subcore_axis_name="s")

    @functools.partial(
        pl.kernel,
        mesh=mesh,
        out_type=jax.ShapeDtypeStruct((n, _HEAD), jnp.float32),
        scratch_types=[
            pltpu.VMEM((chunks, 128), jnp.int32),
            pltpu.VMEM((chunks * 128, _HEAD), jnp.float32),
            pltpu.SemaphoreType.DMA,
        ],
    )
    def gather_k(table_hbm, pos_hbm, out_hbm, idx_v, rows_v, sem):
        wid = lax.axis_index("s") * ncores + lax.axis_index("c")
        row0 = wid * chunks
        pltpu.sync_copy(pos_hbm.at[pl.ds(row0, chunks)], idx_v)
        copies = [
            pltpu.async_copy(
                table_hbm.at[idx_v.at[j]], rows_v.at[pl.ds(j * 128, 128)], sem
            )
            for j in range(chunks)
        ]
        for c in copies:
            c.wait()
        pltpu.sync_copy(rows_v, out_hbm.at[pl.ds(row0 * 128, chunks * 128)])

    return gather_k(table, pos2)


def _tc_apply(cs, query, key, block_t=512):
    n, hidden = query.shape
    heads = hidden // _HEAD

    def body(cs_ref, q_ref, k_ref, oq_ref, ok_ref):
        c = cs_ref[:, :_HALF]
        s = cs_ref[:, _HALF:]
        cosf = jnp.concatenate([c, c], axis=1)
        sinf = jnp.concatenate([-s, s], axis=1)
        for ref, out in ((q_ref, oq_ref), (k_ref, ok_ref)):
            for h in range(heads):
                x = ref[:, h * _HEAD : (h + 1) * _HEAD]
                r = jnp.concatenate([x[:, _HALF:], x[:, :_HALF]], axis=1)
                out[:, h * _HEAD : (h + 1) * _HEAD] = x * cosf + r * sinf

    bs = pl.BlockSpec
    return pl.pallas_call(
        body,
        grid=(n // block_t,),
        in_specs=[
            bs((block_t, _HEAD), lambda i: (i, 0)),
            bs((block_t, hidden), lambda i: (i, 0)),
            bs((block_t, hidden), lambda i: (i, 0)),
        ],
        out_specs=[
            bs((block_t, hidden), lambda i: (i, 0)),
            bs((block_t, hidden), lambda i: (i, 0)),
        ],
        out_shape=[jax.ShapeDtypeStruct((n, hidden), jnp.float32)] * 2,
        compiler_params=pltpu.CompilerParams(dimension_semantics=("parallel",)),
    )(cs, query, key)


def kernel(positions, query, key):
    cs = _sc_gather(_cs_table(), positions)
    q, k = _tc_apply(cs, query, key)
    return (q, k)
